# baseline (speedup 1.0000x reference)
import functools

import jax
import jax.numpy as jnp
from jax import lax
from jax.experimental import pallas as pl
from jax.experimental.pallas import tpu as pltpu
from jax.experimental.pallas import tpu_sc as plsc

N_NODES = 10000
N_EDGES = 320000
IN_DIM = 128
HID_DIM = 32

NC = 2
NS = 16
NW = NC * NS
N_PAD = 10240
STRIPE = N_PAD // NS
E_PER_W = N_EDGES // NW

CHUNK = 80
NCH = E_PER_W // CHUNK
SC_CHUNK = 128
SC_NCH = N_EDGES // SC_CHUNK
SC_BASE_NCH = SC_NCH // NW
SC_MAXCH = SC_BASE_NCH + 1
WAVE = 5
NWAVES = NCH // WAVE
NGRP = 3
NBUF = NGRP * WAVE


def _sc_mesh():
    return plsc.VectorSubcoreMesh(core_axis_name="c", subcore_axis_name="s")


_SC_PARAMS = pltpu.CompilerParams(use_tc_tiling_on_sc=False,
                                  needs_layout_passes=False)


@functools.partial(
    pl.kernel,
    mesh=_sc_mesh(),
    out_type=jax.ShapeDtypeStruct((NC, N_PAD), jnp.float32),
    scratch_types=[
        pltpu.VMEM((SC_MAXCH, SC_CHUNK), jnp.int32),
        pltpu.VMEM((SC_CHUNK,), jnp.float32),
        pltpu.VMEM_SHARED((N_PAD,), jnp.float32),
        pltpu.SemaphoreType.DMA,
    ],
    compiler_params=_SC_PARAMS,
)
def _deg_kernel(ei_hbm, ones_hbm, zeros_hbm, out_hbm, idx_d, ones_v, acc, sem):
    c = lax.axis_index("c")
    s = lax.axis_index("s")
    wid = c * NS + s
    nch = SC_BASE_NCH + jnp.where(wid < 4, 1, 0)
    cb = wid * SC_BASE_NCH + jnp.minimum(wid, 4)
    row0 = pl.multiple_of(s * STRIPE, 8)
    pltpu.sync_copy(zeros_hbm, acc.at[pl.ds(row0, STRIPE)])
    pltpu.sync_copy(ones_hbm, ones_v)
    pltpu.sync_copy(ei_hbm.at[1, pl.ds(cb, SC_BASE_NCH)],
                    idx_d.at[pl.ds(0, SC_BASE_NCH)])

    @pl.when(wid < 4)
    def _():
        pltpu.sync_copy(ei_hbm.at[1, pl.ds(cb + SC_BASE_NCH, 1)],
                        idx_d.at[pl.ds(SC_BASE_NCH, 1)])

    plsc.subcore_barrier()

    def fire(i, carry):
        pltpu.async_copy(ones_v, acc.at[idx_d.at[i]], sem, add=True)
        return carry

    def drain(i, carry):
        pltpu.make_async_copy(ones_v, acc.at[idx_d.at[i]], sem).wait()
        return carry

    lax.fori_loop(0, nch, fire, 0)
    lax.fori_loop(0, nch, drain, 0)
    plsc.subcore_barrier()
    pltpu.sync_copy(acc.at[pl.ds(row0, STRIPE)],
                    out_hbm.at[c, pl.ds(row0, STRIPE)])


@functools.partial(
    pl.kernel,
    mesh=_sc_mesh(),
    out_type=jax.ShapeDtypeStruct((NC, N_PAD, HID_DIM), jnp.float32),
    scratch_types=[
        pltpu.VMEM((NCH, CHUNK), jnp.int32),
        pltpu.VMEM((NCH, CHUNK), jnp.int32),
        pltpu.VMEM((NBUF, CHUNK, HID_DIM), jnp.float32),
        pltpu.VMEM_SHARED((N_PAD, HID_DIM), jnp.float32),
        pltpu.SemaphoreType.DMA,
        pltpu.SemaphoreType.DMA,
    ],
    compiler_params=_SC_PARAMS,
)
def _agg32_kernel(tab_hbm, ei_hbm, zeros_hbm, out_hbm,
                  idx_s, idx_d, rows, acc, sem_g, sem_s):
    c = lax.axis_index("c")
    s = lax.axis_index("s")
    wid = c * NS + s
    row0 = pl.multiple_of(s * STRIPE, 8)
    last_full = (N_NODES - 15 * STRIPE)

    @pl.when((c == 0) & (s < NS - 1))
    def _():
        pltpu.sync_copy(tab_hbm.at[pl.ds(row0, STRIPE)],
                        acc.at[pl.ds(row0, STRIPE)])

    @pl.when((c == 0) & (s == NS - 1))
    def _():
        pltpu.sync_copy(tab_hbm.at[pl.ds(15 * STRIPE, last_full)],
                        acc.at[pl.ds(15 * STRIPE, last_full)])
        pltpu.sync_copy(zeros_hbm.at[pl.ds(0, STRIPE - last_full)],
                        acc.at[pl.ds(15 * STRIPE + last_full,
                                     STRIPE - last_full)])

    @pl.when(c == 1)
    def _():
        pltpu.sync_copy(zeros_hbm, acc.at[pl.ds(row0, STRIPE)])

    pltpu.sync_copy(ei_hbm.at[0, wid], idx_s)
    pltpu.sync_copy(ei_hbm.at[1, wid], idx_d)
    plsc.subcore_barrier()

    def fire_gather(chunk, slot):
        pltpu.async_copy(tab_hbm.at[idx_s.at[chunk]], rows.at[slot], sem_g)

    def drain_gather(chunk, slot):
        pltpu.make_async_copy(tab_hbm.at[idx_s.at[chunk]], rows.at[slot],
                              sem_g).wait()

    def fire_scatter(chunk, slot):
        pltpu.async_copy(rows.at[slot], acc.at[idx_d.at[chunk]], sem_s,
                         add=True)

    def drain_scatter(chunk, slot):
        pltpu.make_async_copy(rows.at[slot], acc.at[idx_d.at[chunk]],
                              sem_s).wait()

    for k in range(WAVE):
        fire_gather(k, k)

    def wave(w, carry):
        g = lax.rem(w, NGRP)
        ng = lax.rem(w + 1, NGRP)
        slot0 = g * WAVE
        nslot0 = ng * WAVE
        base = w * WAVE

        @pl.when(w > 1)
        def _():
            for k in range(WAVE):
                drain_scatter(base - 2 * WAVE + k, nslot0 + k)

        @pl.when(w < NWAVES - 1)
        def _():
            for k in range(WAVE):
                fire_gather(base + WAVE + k, nslot0 + k)

        for k in range(WAVE):
            drain_gather(base + k, slot0 + k)
        for k in range(WAVE):
            fire_scatter(base + k, slot0 + k)
        return carry

    lax.fori_loop(0, NWAVES, wave, 0)
    for w in (NWAVES - 2, NWAVES - 1):
        for k in range(WAVE):
            drain_scatter(w * WAVE + k, (w % NGRP) * WAVE + k)
    plsc.subcore_barrier()
    pltpu.sync_copy(acc.at[pl.ds(row0, STRIPE)],
                    out_hbm.at[c, pl.ds(row0, STRIPE)])


@functools.partial(
    pl.kernel,
    mesh=_sc_mesh(),
    out_type=jax.ShapeDtypeStruct((NC, N_PAD), jnp.float32),
    scratch_types=[
        pltpu.VMEM((SC_MAXCH, SC_CHUNK), jnp.int32),
        pltpu.VMEM((SC_MAXCH, SC_CHUNK), jnp.int32),
        pltpu.VMEM((N_PAD,), jnp.float32),
        pltpu.VMEM((SC_MAXCH * SC_CHUNK,), jnp.float32),
        pltpu.VMEM_SHARED((N_PAD,), jnp.float32),
        pltpu.SemaphoreType.DMA,
    ],
    compiler_params=_SC_PARAMS,
)
def _agg1_kernel(tab_hbm, ei_hbm, zeros_hbm, out_hbm,
                 idx_s, idx_d, tab_v, vals, acc, sem):
    c = lax.axis_index("c")
    s = lax.axis_index("s")
    wid = c * NS + s
    nch = SC_BASE_NCH + jnp.where(wid < 4, 1, 0)
    cb = wid * SC_BASE_NCH + jnp.minimum(wid, 4)
    row0 = pl.multiple_of(s * STRIPE, 8)
    pltpu.sync_copy(zeros_hbm, acc.at[pl.ds(row0, STRIPE)])
    pltpu.sync_copy(tab_hbm, tab_v)
    pltpu.sync_copy(ei_hbm.at[0, pl.ds(cb, SC_BASE_NCH)],
                    idx_s.at[pl.ds(0, SC_BASE_NCH)])
    pltpu.sync_copy(ei_hbm.at[1, pl.ds(cb, SC_BASE_NCH)],
                    idx_d.at[pl.ds(0, SC_BASE_NCH)])

    @pl.when(wid < 4)
    def _():
        pltpu.sync_copy(ei_hbm.at[0, pl.ds(cb + SC_BASE_NCH, 1)],
                        idx_s.at[pl.ds(SC_BASE_NCH, 1)])
        pltpu.sync_copy(ei_hbm.at[1, pl.ds(cb + SC_BASE_NCH, 1)],
                        idx_d.at[pl.ds(SC_BASE_NCH, 1)])

    plsc.subcore_barrier()

    def step(i, carry):
        for j in range(SC_CHUNK // 16):
            iv = idx_s[i, pl.ds(j * 16, 16)]
            vals[pl.ds(i * SC_CHUNK + j * 16, 16)] = plsc.load_gather(
                tab_v, [iv])
        pltpu.async_copy(vals.at[pl.ds(i * SC_CHUNK, SC_CHUNK)],
                         acc.at[idx_d.at[i]], sem, add=True)
        return carry

    def drain(i, carry):
        pltpu.make_async_copy(vals.at[pl.ds(i * SC_CHUNK, SC_CHUNK)],
                              acc.at[idx_d.at[i]], sem).wait()
        return carry

    lax.fori_loop(0, nch, step, 0)
    lax.fori_loop(0, nch, drain, 0)
    plsc.subcore_barrier()
    pltpu.sync_copy(acc.at[pl.ds(row0, STRIPE)],
                    out_hbm.at[c, pl.ds(row0, STRIPE)])


def _t1_body(x_ref, w1_ref, degp_ref, y1_ref):
    deg = degp_ref[0, :N_NODES] + degp_ref[1, :N_NODES] + 1.0
    dinv = lax.rsqrt(deg)
    h = jnp.dot(x_ref[...], w1_ref[...], preferred_element_type=jnp.float32)
    y1_ref[...] = h * dinv[:, None]


_t1 = pl.pallas_call(
    _t1_body,
    out_shape=jax.ShapeDtypeStruct((N_NODES, HID_DIM), jnp.float32),
)


_PROW = N_PAD // 4


def _t2_body(aggp_ref, degp_ref, b1p_ref, w2b_ref, y2_ref):
    tot = aggp_ref[:_PROW, :] + aggp_ref[_PROW:, :]
    deg4 = degp_ref[0] + degp_ref[1] + 1.0
    dinv4 = lax.rsqrt(deg4)
    scale = jnp.concatenate(
        [jnp.broadcast_to(dinv4[:, k:k + 1], (_PROW, HID_DIM))
         for k in range(4)], axis=1)
    x2 = jnp.maximum(scale * tot + b1p_ref[...], 0.0)
    g4 = jnp.dot(x2, w2b_ref[...], preferred_element_type=jnp.float32)
    y2_ref[...] = dinv4 * g4


_t2 = pl.pallas_call(
    _t2_body,
    out_shape=jax.ShapeDtypeStruct((_PROW, 4), jnp.float32),
)


_PR128 = N_PAD // 128


def _t3_body(agg2p_ref, y2p_ref, degp_ref, b2_ref, out_ref):
    deg = degp_ref[:_PR128, :] + degp_ref[_PR128:, :] + 1.0
    dinv = lax.rsqrt(deg)
    a = agg2p_ref[:_PR128, :] + agg2p_ref[_PR128:, :] + y2p_ref[...]
    out_ref[...] = dinv * a + b2_ref[0, 0]


_t3 = pl.pallas_call(
    _t3_body,
    out_shape=jax.ShapeDtypeStruct((_PR128, 128), jnp.float32),
)


def kernel(x, edge_index, W1, b1, W2, b2):
    ei4 = edge_index.reshape(2, NW, NCH, CHUNK)
    ei128 = edge_index.reshape(2, SC_NCH, SC_CHUNK)

    zeros32 = jnp.zeros((STRIPE, HID_DIM), jnp.float32)
    zeros1 = jnp.zeros((STRIPE,), jnp.float32)
    ones_c = jnp.ones((SC_CHUNK,), jnp.float32)

    b1p = jnp.tile(b1, 4).reshape(1, 4 * HID_DIM)
    w2b = jnp.zeros((4 * HID_DIM, 4), jnp.float32)
    for k in range(4):
        w2b = w2b.at[k * HID_DIM:(k + 1) * HID_DIM, k].set(W2[:, 0])

    degp = _deg_kernel(ei128, ones_c, zeros1)
    y1 = _t1(x, W1, degp)
    aggp = _agg32_kernel(y1, ei4, zeros32)
    y2_4 = _t2(aggp.reshape(2 * _PROW, 4 * HID_DIM),
               degp.reshape(2, _PROW, 4), b1p, w2b)
    y2 = y2_4.reshape(-1)
    agg2p = _agg1_kernel(y2, ei128, zeros1)
    outp = _t3(agg2p.reshape(2 * _PR128, 128), y2.reshape(_PR128, 128),
               degp.reshape(2 * _PR128, 128), b2.reshape(1, 1))
    return outp.reshape(-1)[:N_NODES]


# speedup vs baseline: 1.0367x; 1.0367x over previous
import functools

import jax
import jax.numpy as jnp
from jax import lax
from jax.experimental import pallas as pl
from jax.experimental.pallas import tpu as pltpu
from jax.experimental.pallas import tpu_sc as plsc

N_NODES = 10000
N_EDGES = 320000
IN_DIM = 128
HID_DIM = 32

NC = 2
NS = 16
NW = NC * NS
N_PAD = 10240
STRIPE = N_PAD // NS
E_PER_W = N_EDGES // NW

CHUNK = 80
NCH = E_PER_W // CHUNK
WAVE = 5
NWAVES = NCH // WAVE
NGRP = 3
NBUF = NGRP * WAVE


def _sc_mesh():
    return plsc.VectorSubcoreMesh(core_axis_name="c", subcore_axis_name="s")


_SC_PARAMS = pltpu.CompilerParams(use_tc_tiling_on_sc=False,
                                  needs_layout_passes=False)


@functools.partial(
    pl.kernel,
    mesh=_sc_mesh(),
    out_type=jax.ShapeDtypeStruct((NC, N_PAD), jnp.float32),
    scratch_types=[
        pltpu.VMEM((NCH, CHUNK), jnp.int32),
        pltpu.VMEM((CHUNK,), jnp.float32),
        pltpu.VMEM_SHARED((N_PAD,), jnp.float32),
        pltpu.SemaphoreType.DMA,
    ],
    compiler_params=_SC_PARAMS,
)
def _deg_kernel(ei_hbm, ones_hbm, zeros_hbm, out_hbm, idx_d, ones_v, acc, sem):
    c = lax.axis_index("c")
    s = lax.axis_index("s")
    wid = c * NS + s
    row0 = pl.multiple_of(s * STRIPE, 8)
    pltpu.sync_copy(zeros_hbm, acc.at[pl.ds(row0, STRIPE)])
    pltpu.sync_copy(ones_hbm, ones_v)
    pltpu.sync_copy(ei_hbm.at[1, wid], idx_d)
    plsc.subcore_barrier()

    def fire(i, carry):
        pltpu.async_copy(ones_v, acc.at[idx_d.at[i]], sem, add=True)
        return carry

    def drain(i, carry):
        pltpu.make_async_copy(ones_v, acc.at[idx_d.at[i]], sem).wait()
        return carry

    lax.fori_loop(0, NCH, fire, 0)
    lax.fori_loop(0, NCH, drain, 0)
    plsc.subcore_barrier()
    pltpu.sync_copy(acc.at[pl.ds(row0, STRIPE)],
                    out_hbm.at[c, pl.ds(row0, STRIPE)])


@functools.partial(
    pl.kernel,
    mesh=_sc_mesh(),
    out_type=jax.ShapeDtypeStruct((NC, N_PAD, HID_DIM), jnp.float32),
    scratch_types=[
        pltpu.VMEM((NCH, CHUNK), jnp.int32),
        pltpu.VMEM((NCH, CHUNK), jnp.int32),
        pltpu.VMEM((NBUF, CHUNK, HID_DIM), jnp.float32),
        pltpu.VMEM_SHARED((N_PAD, HID_DIM), jnp.float32),
        pltpu.SemaphoreType.DMA,
        pltpu.SemaphoreType.DMA,
    ],
    compiler_params=_SC_PARAMS,
)
def _agg32_kernel(tab_hbm, ei_hbm, zeros_hbm, out_hbm,
                  idx_s, idx_d, rows, acc, sem_g, sem_s):
    c = lax.axis_index("c")
    s = lax.axis_index("s")
    wid = c * NS + s
    row0 = pl.multiple_of(s * STRIPE, 8)
    last_full = (N_NODES - 15 * STRIPE)

    @pl.when((c == 0) & (s < NS - 1))
    def _():
        pltpu.sync_copy(tab_hbm.at[pl.ds(row0, STRIPE)],
                        acc.at[pl.ds(row0, STRIPE)])

    @pl.when((c == 0) & (s == NS - 1))
    def _():
        pltpu.sync_copy(tab_hbm.at[pl.ds(15 * STRIPE, last_full)],
                        acc.at[pl.ds(15 * STRIPE, last_full)])
        pltpu.sync_copy(zeros_hbm.at[pl.ds(0, STRIPE - last_full)],
                        acc.at[pl.ds(15 * STRIPE + last_full,
                                     STRIPE - last_full)])

    @pl.when(c == 1)
    def _():
        pltpu.sync_copy(zeros_hbm, acc.at[pl.ds(row0, STRIPE)])

    pltpu.sync_copy(ei_hbm.at[0, wid], idx_s)
    pltpu.sync_copy(ei_hbm.at[1, wid], idx_d)
    plsc.subcore_barrier()

    def fire_gather(chunk, slot):
        pltpu.async_copy(tab_hbm.at[idx_s.at[chunk]], rows.at[slot], sem_g)

    def drain_gather(chunk, slot):
        pltpu.make_async_copy(tab_hbm.at[idx_s.at[chunk]], rows.at[slot],
                              sem_g).wait()

    def fire_scatter(chunk, slot):
        pltpu.async_copy(rows.at[slot], acc.at[idx_d.at[chunk]], sem_s,
                         add=True)

    def drain_scatter(chunk, slot):
        pltpu.make_async_copy(rows.at[slot], acc.at[idx_d.at[chunk]],
                              sem_s).wait()

    for k in range(WAVE):
        fire_gather(k, k)

    def wave(w, carry):
        g = lax.rem(w, NGRP)
        ng = lax.rem(w + 1, NGRP)
        slot0 = g * WAVE
        nslot0 = ng * WAVE
        base = w * WAVE

        @pl.when(w > 1)
        def _():
            for k in range(WAVE):
                drain_scatter(base - 2 * WAVE + k, nslot0 + k)

        @pl.when(w < NWAVES - 1)
        def _():
            for k in range(WAVE):
                fire_gather(base + WAVE + k, nslot0 + k)

        for k in range(WAVE):
            drain_gather(base + k, slot0 + k)
        for k in range(WAVE):
            fire_scatter(base + k, slot0 + k)
        return carry

    lax.fori_loop(0, NWAVES, wave, 0)
    for w in (NWAVES - 2, NWAVES - 1):
        for k in range(WAVE):
            drain_scatter(w * WAVE + k, (w % NGRP) * WAVE + k)
    plsc.subcore_barrier()
    pltpu.sync_copy(acc.at[pl.ds(row0, STRIPE)],
                    out_hbm.at[c, pl.ds(row0, STRIPE)])


@functools.partial(
    pl.kernel,
    mesh=_sc_mesh(),
    out_type=jax.ShapeDtypeStruct((NC, N_PAD), jnp.float32),
    scratch_types=[
        pltpu.VMEM((NCH, CHUNK), jnp.int32),
        pltpu.VMEM((NCH, CHUNK), jnp.int32),
        pltpu.VMEM((N_PAD,), jnp.float32),
        pltpu.VMEM((E_PER_W,), jnp.float32),
        pltpu.VMEM_SHARED((N_PAD,), jnp.float32),
        pltpu.SemaphoreType.DMA,
    ],
    compiler_params=_SC_PARAMS,
)
def _agg1_kernel(tab_hbm, ei_hbm, zeros_hbm, out_hbm,
                 idx_s, idx_d, tab_v, vals, acc, sem):
    c = lax.axis_index("c")
    s = lax.axis_index("s")
    wid = c * NS + s
    row0 = pl.multiple_of(s * STRIPE, 8)
    pltpu.sync_copy(zeros_hbm, acc.at[pl.ds(row0, STRIPE)])
    pltpu.sync_copy(tab_hbm, tab_v)
    pltpu.sync_copy(ei_hbm.at[0, wid], idx_s)
    pltpu.sync_copy(ei_hbm.at[1, wid], idx_d)
    plsc.subcore_barrier()

    def step(i, carry):
        for j in range(CHUNK // 16):
            iv = idx_s[i, pl.ds(j * 16, 16)]
            vals[pl.ds(i * CHUNK + j * 16, 16)] = plsc.load_gather(
                tab_v, [iv])
        pltpu.async_copy(vals.at[pl.ds(i * CHUNK, CHUNK)],
                         acc.at[idx_d.at[i]], sem, add=True)
        return carry

    def drain(i, carry):
        pltpu.make_async_copy(vals.at[pl.ds(i * CHUNK, CHUNK)],
                              acc.at[idx_d.at[i]], sem).wait()
        return carry

    lax.fori_loop(0, NCH, step, 0)
    lax.fori_loop(0, NCH, drain, 0)
    plsc.subcore_barrier()
    pltpu.sync_copy(acc.at[pl.ds(row0, STRIPE)],
                    out_hbm.at[c, pl.ds(row0, STRIPE)])


def _t1_body(x_ref, w1_ref, degp_ref, y1_ref):
    deg = degp_ref[0, :N_NODES] + degp_ref[1, :N_NODES] + 1.0
    dinv = lax.rsqrt(deg)
    h = jnp.dot(x_ref[...], w1_ref[...], preferred_element_type=jnp.float32)
    y1_ref[...] = h * dinv[:, None]


_t1 = pl.pallas_call(
    _t1_body,
    out_shape=jax.ShapeDtypeStruct((N_NODES, HID_DIM), jnp.float32),
)


_PROW = N_PAD // 4


def _t2_body(aggp_ref, degp_ref, b1p_ref, w2b_ref, y2_ref):
    tot = aggp_ref[:_PROW, :] + aggp_ref[_PROW:, :]
    deg4 = degp_ref[0] + degp_ref[1] + 1.0
    dinv4 = lax.rsqrt(deg4)
    scale = jnp.concatenate(
        [jnp.broadcast_to(dinv4[:, k:k + 1], (_PROW, HID_DIM))
         for k in range(4)], axis=1)
    x2 = jnp.maximum(scale * tot + b1p_ref[...], 0.0)
    g4 = jnp.dot(x2, w2b_ref[...], preferred_element_type=jnp.float32)
    y2_ref[...] = dinv4 * g4


_t2 = pl.pallas_call(
    _t2_body,
    out_shape=jax.ShapeDtypeStruct((_PROW, 4), jnp.float32),
)


_PR128 = N_PAD // 128


def _t3_body(agg2p_ref, y2p_ref, degp_ref, b2_ref, out_ref):
    deg = degp_ref[:_PR128, :] + degp_ref[_PR128:, :] + 1.0
    dinv = lax.rsqrt(deg)
    a = agg2p_ref[:_PR128, :] + agg2p_ref[_PR128:, :] + y2p_ref[...]
    out_ref[...] = dinv * a + b2_ref[0, 0]


_t3 = pl.pallas_call(
    _t3_body,
    out_shape=jax.ShapeDtypeStruct((_PR128, 128), jnp.float32),
)


def kernel(x, edge_index, W1, b1, W2, b2):
    ei4 = edge_index.reshape(2, NW, NCH, CHUNK)

    zeros32 = jnp.zeros((STRIPE, HID_DIM), jnp.float32)
    zeros1 = jnp.zeros((STRIPE,), jnp.float32)
    ones_c = jnp.ones((CHUNK,), jnp.float32)

    b1p = jnp.tile(b1, 4).reshape(1, 4 * HID_DIM)
    w2b = jnp.zeros((4 * HID_DIM, 4), jnp.float32)
    for k in range(4):
        w2b = w2b.at[k * HID_DIM:(k + 1) * HID_DIM, k].set(W2[:, 0])

    degp = _deg_kernel(ei4, ones_c, zeros1)
    y1 = _t1(x, W1, degp)
    aggp = _agg32_kernel(y1, ei4, zeros32)
    y2_4 = _t2(aggp.reshape(2 * _PROW, 4 * HID_DIM),
               degp.reshape(2, _PROW, 4), b1p, w2b)
    y2 = y2_4.reshape(-1)
    agg2p = _agg1_kernel(y2, ei4, zeros1)
    outp = _t3(agg2p.reshape(2 * _PR128, 128), y2.reshape(_PR128, 128),
               degp.reshape(2 * _PR128, 128), b2.reshape(1, 1))
    return outp.reshape(-1)[:N_NODES]
